# baseline (speedup 1.0000x reference)
import functools
import jax
import jax.numpy as jnp
from jax import lax
from jax.experimental import pallas as pl

_SQRT_HALF = 0.7071067811865476
_F32_EPS = float(jnp.finfo(jnp.float32).eps)


def _gelu(x):
    return 0.5 * x * (1.0 + lax.erf(x * _SQRT_HALF))


def _proj_kernel(sfeat_ref, w1b_ref, p_ref):
    p = lax.dot(sfeat_ref[0], w1b_ref[...],
                preferred_element_type=jnp.float32)
    p_ref[0] = p.astype(jnp.bfloat16)


def _topk_interp_kernel(xyz_ref, pf_ref, sxyzt_ref, p_ref, w1t_ref,
                        h1_ref, s1_ref, q1_ref, *, nb, s_pts):
    b = pl.program_id(0)
    n = pl.program_id(1)

    x = xyz_ref[0]
    st = sxyzt_ref[0]
    t = lax.dot(x.astype(jnp.bfloat16), st.astype(jnp.bfloat16),
                preferred_element_type=jnp.float32)
    xn = jnp.sum(x * x, axis=1, keepdims=True)
    dd = jnp.sum(st * st, axis=0, keepdims=True) - 2.0 * t

    p = p_ref[0]
    acc = jnp.zeros((nb, p.shape[1]), jnp.float32)
    wsum = jnp.zeros((nb, 1), jnp.float32)
    for _ in range(3):
        m = jnp.min(dd, axis=1, keepdims=True)
        e = dd == m
        w = 1.0 / (jnp.maximum(m + xn, 0.0) + _F32_EPS)
        oh = jnp.where(e, 1.0, 0.0).astype(jnp.bfloat16)
        row = lax.dot(oh, p, preferred_element_type=jnp.float32)
        acc = acc + w * row
        wsum = wsum + w
        dd = jnp.where(e, jnp.float32(jnp.inf), dd)

    h1 = acc / wsum
    h1 = h1 + lax.dot(pf_ref[0].astype(jnp.bfloat16), w1t_ref[...],
                      preferred_element_type=jnp.float32)
    h1_ref[0] = h1.astype(jnp.bfloat16)

    @pl.when((b == 0) & (n == 0))
    def _init():
        s1_ref[...] = jnp.zeros_like(s1_ref)
        q1_ref[...] = jnp.zeros_like(q1_ref)

    s1_ref[...] += jnp.sum(h1.reshape(nb // 8, 8, h1.shape[1]), axis=0)
    q1_ref[...] += jnp.sum((h1 * h1).reshape(nb // 8, 8, h1.shape[1]), axis=0)


def _bn_gelu_mm_kernel(h_ref, s_ref, q_ref, g_ref, bt_ref, w_ref,
                       out_ref, s2_ref, q2_ref, *, count, nb):
    i = pl.program_id(0)
    mean = jnp.sum(s_ref[...], axis=0, keepdims=True) / count
    var = jnp.sum(q_ref[...], axis=0, keepdims=True) / count - mean * mean
    scale = g_ref[...] * lax.rsqrt(var + 1e-5)
    x = h_ref[...].astype(jnp.float32)
    xn = (x - mean) * scale + bt_ref[...]
    g = _gelu(xn).astype(jnp.bfloat16)
    h2 = lax.dot(g, w_ref[...], preferred_element_type=jnp.float32)
    out_ref[...] = h2.astype(jnp.bfloat16)

    @pl.when(i == 0)
    def _init():
        s2_ref[...] = jnp.zeros_like(s2_ref)
        q2_ref[...] = jnp.zeros_like(q2_ref)

    s2_ref[...] += jnp.sum(h2.reshape(nb // 8, 8, h2.shape[1]), axis=0)
    q2_ref[...] += jnp.sum((h2 * h2).reshape(nb // 8, 8, h2.shape[1]), axis=0)


def _bn_gelu_kernel(h_ref, s_ref, q_ref, g_ref, bt_ref, out_ref, *, count):
    mean = jnp.sum(s_ref[...], axis=0, keepdims=True) / count
    var = jnp.sum(q_ref[...], axis=0, keepdims=True) / count - mean * mean
    scale = g_ref[...] * lax.rsqrt(var + 1e-5)
    x = h_ref[...].astype(jnp.float32)
    out_ref[...] = _gelu((x - mean) * scale + bt_ref[...])


def kernel(super_xyz, super_point_features, xyz, point_features,
           W1, gamma1, beta1, W2, gamma2, beta2):
    B, S, F = super_point_features.shape
    N = xyz.shape[1]
    C = point_features.shape[2]
    H1 = W1.shape[1]
    H2 = W2.shape[1]
    NB = 512
    M = B * N
    NB2 = 2048

    sxyzt = jnp.transpose(super_xyz, (0, 2, 1))
    w1_top = W1[:C].astype(jnp.bfloat16)
    w1_bot = W1[C:]
    w2_b = W2.astype(jnp.bfloat16)

    P = pl.pallas_call(
        _proj_kernel,
        grid=(B,),
        in_specs=[
            pl.BlockSpec((1, S, F), lambda b: (b, 0, 0)),
            pl.BlockSpec((F, H1), lambda b: (0, 0)),
        ],
        out_specs=pl.BlockSpec((1, S, H1), lambda b: (b, 0, 0)),
        out_shape=jax.ShapeDtypeStruct((B, S, H1), jnp.bfloat16),
    )(super_point_features, w1_bot)

    h1, s1, q1 = pl.pallas_call(
        functools.partial(_topk_interp_kernel, nb=NB, s_pts=S),
        grid=(B, N // NB),
        in_specs=[
            pl.BlockSpec((1, NB, 3), lambda b, n: (b, n, 0)),
            pl.BlockSpec((1, NB, C), lambda b, n: (b, n, 0)),
            pl.BlockSpec((1, 3, S), lambda b, n: (b, 0, 0)),
            pl.BlockSpec((1, S, H1), lambda b, n: (b, 0, 0)),
            pl.BlockSpec((C, H1), lambda b, n: (0, 0)),
        ],
        out_specs=[
            pl.BlockSpec((1, NB, H1), lambda b, n: (b, n, 0)),
            pl.BlockSpec((8, H1), lambda b, n: (0, 0)),
            pl.BlockSpec((8, H1), lambda b, n: (0, 0)),
        ],
        out_shape=[
            jax.ShapeDtypeStruct((B, N, H1), jnp.bfloat16),
            jax.ShapeDtypeStruct((8, H1), jnp.float32),
            jax.ShapeDtypeStruct((8, H1), jnp.float32),
        ],
    )(xyz, point_features, sxyzt, P, w1_top)

    h1f = h1.reshape(M, H1)
    h2, s2, q2 = pl.pallas_call(
        functools.partial(_bn_gelu_mm_kernel, count=float(M), nb=NB2),
        grid=(M // NB2,),
        in_specs=[
            pl.BlockSpec((NB2, H1), lambda i: (i, 0)),
            pl.BlockSpec((8, H1), lambda i: (0, 0)),
            pl.BlockSpec((8, H1), lambda i: (0, 0)),
            pl.BlockSpec((1, H1), lambda i: (0, 0)),
            pl.BlockSpec((1, H1), lambda i: (0, 0)),
            pl.BlockSpec((H1, H2), lambda i: (0, 0)),
        ],
        out_specs=[
            pl.BlockSpec((NB2, H2), lambda i: (i, 0)),
            pl.BlockSpec((8, H2), lambda i: (0, 0)),
            pl.BlockSpec((8, H2), lambda i: (0, 0)),
        ],
        out_shape=[
            jax.ShapeDtypeStruct((M, H2), jnp.bfloat16),
            jax.ShapeDtypeStruct((8, H2), jnp.float32),
            jax.ShapeDtypeStruct((8, H2), jnp.float32),
        ],
    )(h1f, s1, q1, gamma1.reshape(1, H1), beta1.reshape(1, H1), w2_b)

    out = pl.pallas_call(
        functools.partial(_bn_gelu_kernel, count=float(M)),
        grid=(M // NB2,),
        in_specs=[
            pl.BlockSpec((NB2, H2), lambda i: (i, 0)),
            pl.BlockSpec((8, H2), lambda i: (0, 0)),
            pl.BlockSpec((8, H2), lambda i: (0, 0)),
            pl.BlockSpec((1, H2), lambda i: (0, 0)),
            pl.BlockSpec((1, H2), lambda i: (0, 0)),
        ],
        out_specs=pl.BlockSpec((NB2, H2), lambda i: (i, 0)),
        out_shape=jax.ShapeDtypeStruct((M, H2), jnp.float32),
    )(h2, s2, q2, gamma2.reshape(1, H2), beta2.reshape(1, H2))

    return out.reshape(B, N, H2)


# speedup vs baseline: 1.0458x; 1.0458x over previous
import functools
import jax
import jax.numpy as jnp
from jax import lax
from jax.experimental import pallas as pl

_SQRT_HALF = 0.7071067811865476
_F32_EPS = float(jnp.finfo(jnp.float32).eps)


def _gelu(x):
    return 0.5 * x * (1.0 + lax.erf(x * _SQRT_HALF))


def _proj_kernel(sfeat_ref, w1b_ref, p_ref):
    p = lax.dot(sfeat_ref[0], w1b_ref[...],
                preferred_element_type=jnp.float32)
    p_ref[0] = p.astype(jnp.bfloat16)


def _topk_interp_kernel(xyz_ref, pf_ref, sxyzt_ref, p_ref, w1t_ref,
                        h1_ref, s1_ref, q1_ref, *, nb, s_pts):
    b = pl.program_id(0)
    n = pl.program_id(1)

    x = xyz_ref[0]
    st = sxyzt_ref[0]
    t = lax.dot(x.astype(jnp.bfloat16), st.astype(jnp.bfloat16),
                preferred_element_type=jnp.float32)
    xn = jnp.sum(x * x, axis=1, keepdims=True)
    dd = jnp.sum(st * st, axis=0, keepdims=True) - 2.0 * t

    wmat = jnp.zeros_like(dd)
    wsum = jnp.zeros((nb, 1), jnp.float32)
    for _ in range(3):
        m = jnp.min(dd, axis=1, keepdims=True)
        e = dd == m
        w = 1.0 / (jnp.maximum(m + xn, 0.0) + _F32_EPS)
        wmat = jnp.where(e, jnp.broadcast_to(w, dd.shape), wmat)
        wsum = wsum + w
        dd = jnp.where(e, jnp.float32(jnp.inf), dd)
    wmat = (wmat / wsum).astype(jnp.bfloat16)

    h1 = lax.dot(wmat, p_ref[0], preferred_element_type=jnp.float32)
    h1 = h1 + lax.dot(pf_ref[0].astype(jnp.bfloat16), w1t_ref[...],
                      preferred_element_type=jnp.float32)
    h1_ref[0] = h1.astype(jnp.bfloat16)

    @pl.when((b == 0) & (n == 0))
    def _init():
        s1_ref[...] = jnp.zeros_like(s1_ref)
        q1_ref[...] = jnp.zeros_like(q1_ref)

    s1_ref[...] += jnp.sum(h1.reshape(nb // 8, 8, h1.shape[1]), axis=0)
    q1_ref[...] += jnp.sum((h1 * h1).reshape(nb // 8, 8, h1.shape[1]), axis=0)


def _bn_gelu_mm_kernel(h_ref, s_ref, q_ref, g_ref, bt_ref, w_ref,
                       out_ref, s2_ref, q2_ref, *, count, nb):
    i = pl.program_id(0)
    mean = jnp.sum(s_ref[...], axis=0, keepdims=True) / count
    var = jnp.sum(q_ref[...], axis=0, keepdims=True) / count - mean * mean
    scale = g_ref[...] * lax.rsqrt(var + 1e-5)
    x = h_ref[...].astype(jnp.float32)
    xn = (x - mean) * scale + bt_ref[...]
    g = _gelu(xn).astype(jnp.bfloat16)
    h2 = lax.dot(g, w_ref[...], preferred_element_type=jnp.float32)
    out_ref[...] = h2.astype(jnp.bfloat16)

    @pl.when(i == 0)
    def _init():
        s2_ref[...] = jnp.zeros_like(s2_ref)
        q2_ref[...] = jnp.zeros_like(q2_ref)

    s2_ref[...] += jnp.sum(h2.reshape(nb // 8, 8, h2.shape[1]), axis=0)
    q2_ref[...] += jnp.sum((h2 * h2).reshape(nb // 8, 8, h2.shape[1]), axis=0)


def _bn_gelu_kernel(h_ref, s_ref, q_ref, g_ref, bt_ref, out_ref, *, count):
    mean = jnp.sum(s_ref[...], axis=0, keepdims=True) / count
    var = jnp.sum(q_ref[...], axis=0, keepdims=True) / count - mean * mean
    scale = g_ref[...] * lax.rsqrt(var + 1e-5)
    x = h_ref[...].astype(jnp.float32)
    out_ref[...] = _gelu((x - mean) * scale + bt_ref[...])


def kernel(super_xyz, super_point_features, xyz, point_features,
           W1, gamma1, beta1, W2, gamma2, beta2):
    B, S, F = super_point_features.shape
    N = xyz.shape[1]
    C = point_features.shape[2]
    H1 = W1.shape[1]
    H2 = W2.shape[1]
    NB = 512
    M = B * N
    NB2 = 2048

    sxyzt = jnp.transpose(super_xyz, (0, 2, 1))
    w1_top = W1[:C].astype(jnp.bfloat16)
    w1_bot = W1[C:]
    w2_b = W2.astype(jnp.bfloat16)

    P = pl.pallas_call(
        _proj_kernel,
        grid=(B,),
        in_specs=[
            pl.BlockSpec((1, S, F), lambda b: (b, 0, 0)),
            pl.BlockSpec((F, H1), lambda b: (0, 0)),
        ],
        out_specs=pl.BlockSpec((1, S, H1), lambda b: (b, 0, 0)),
        out_shape=jax.ShapeDtypeStruct((B, S, H1), jnp.bfloat16),
    )(super_point_features, w1_bot)

    h1, s1, q1 = pl.pallas_call(
        functools.partial(_topk_interp_kernel, nb=NB, s_pts=S),
        grid=(B, N // NB),
        in_specs=[
            pl.BlockSpec((1, NB, 3), lambda b, n: (b, n, 0)),
            pl.BlockSpec((1, NB, C), lambda b, n: (b, n, 0)),
            pl.BlockSpec((1, 3, S), lambda b, n: (b, 0, 0)),
            pl.BlockSpec((1, S, H1), lambda b, n: (b, 0, 0)),
            pl.BlockSpec((C, H1), lambda b, n: (0, 0)),
        ],
        out_specs=[
            pl.BlockSpec((1, NB, H1), lambda b, n: (b, n, 0)),
            pl.BlockSpec((8, H1), lambda b, n: (0, 0)),
            pl.BlockSpec((8, H1), lambda b, n: (0, 0)),
        ],
        out_shape=[
            jax.ShapeDtypeStruct((B, N, H1), jnp.bfloat16),
            jax.ShapeDtypeStruct((8, H1), jnp.float32),
            jax.ShapeDtypeStruct((8, H1), jnp.float32),
        ],
    )(xyz, point_features, sxyzt, P, w1_top)

    h1f = h1.reshape(M, H1)
    h2, s2, q2 = pl.pallas_call(
        functools.partial(_bn_gelu_mm_kernel, count=float(M), nb=NB2),
        grid=(M // NB2,),
        in_specs=[
            pl.BlockSpec((NB2, H1), lambda i: (i, 0)),
            pl.BlockSpec((8, H1), lambda i: (0, 0)),
            pl.BlockSpec((8, H1), lambda i: (0, 0)),
            pl.BlockSpec((1, H1), lambda i: (0, 0)),
            pl.BlockSpec((1, H1), lambda i: (0, 0)),
            pl.BlockSpec((H1, H2), lambda i: (0, 0)),
        ],
        out_specs=[
            pl.BlockSpec((NB2, H2), lambda i: (i, 0)),
            pl.BlockSpec((8, H2), lambda i: (0, 0)),
            pl.BlockSpec((8, H2), lambda i: (0, 0)),
        ],
        out_shape=[
            jax.ShapeDtypeStruct((M, H2), jnp.bfloat16),
            jax.ShapeDtypeStruct((8, H2), jnp.float32),
            jax.ShapeDtypeStruct((8, H2), jnp.float32),
        ],
    )(h1f, s1, q1, gamma1.reshape(1, H1), beta1.reshape(1, H1), w2_b)

    out = pl.pallas_call(
        functools.partial(_bn_gelu_kernel, count=float(M)),
        grid=(M // NB2,),
        in_specs=[
            pl.BlockSpec((NB2, H2), lambda i: (i, 0)),
            pl.BlockSpec((8, H2), lambda i: (0, 0)),
            pl.BlockSpec((8, H2), lambda i: (0, 0)),
            pl.BlockSpec((1, H2), lambda i: (0, 0)),
            pl.BlockSpec((1, H2), lambda i: (0, 0)),
        ],
        out_specs=pl.BlockSpec((NB2, H2), lambda i: (i, 0)),
        out_shape=jax.ShapeDtypeStruct((M, H2), jnp.float32),
    )(h2, s2, q2, gamma2.reshape(1, H2), beta2.reshape(1, H2))

    return out.reshape(B, N, H2)


# speedup vs baseline: 1.0587x; 1.0123x over previous
import functools
import jax
import jax.numpy as jnp
from jax import lax
from jax.experimental import pallas as pl
from jax.experimental.pallas import tpu as pltpu

_SQRT_HALF = 0.7071067811865476
_F32_EPS = float(jnp.finfo(jnp.float32).eps)


def _gelu(x):
    return 0.5 * x * (1.0 + lax.erf(x * _SQRT_HALF))


def _topk_interp_kernel(xyz_ref, pf_ref, sxyzt_ref, sfeat_ref, w1b_ref,
                        w1t_ref, h1_ref, s1_ref, q1_ref, p_scr, *, nb):
    b = pl.program_id(0)
    n = pl.program_id(1)

    @pl.when(n == 0)
    def _proj():
        p_scr[...] = lax.dot(
            sfeat_ref[0], w1b_ref[...],
            preferred_element_type=jnp.float32).astype(jnp.bfloat16)

    x = xyz_ref[0]
    st = sxyzt_ref[0]
    t = lax.dot(x.astype(jnp.bfloat16), st.astype(jnp.bfloat16),
                preferred_element_type=jnp.float32)
    xn = jnp.sum(x * x, axis=1, keepdims=True)
    dd = jnp.sum(st * st, axis=0, keepdims=True) - 2.0 * t

    wmat = jnp.zeros_like(dd)
    wsum = jnp.zeros((nb, 1), jnp.float32)
    for _ in range(3):
        m = jnp.min(dd, axis=1, keepdims=True)
        e = dd == m
        w = 1.0 / (jnp.maximum(m + xn, 0.0) + _F32_EPS)
        wmat = jnp.where(e, jnp.broadcast_to(w, dd.shape), wmat)
        wsum = wsum + w
        dd = jnp.where(e, jnp.float32(jnp.inf), dd)
    wmat = (wmat / wsum).astype(jnp.bfloat16)

    h1 = lax.dot(wmat, p_scr[...], preferred_element_type=jnp.float32)
    h1 = h1 + lax.dot(pf_ref[0].astype(jnp.bfloat16), w1t_ref[...],
                      preferred_element_type=jnp.float32)
    h1_ref[0] = h1.astype(jnp.bfloat16)

    @pl.when((b == 0) & (n == 0))
    def _init():
        s1_ref[...] = jnp.zeros_like(s1_ref)
        q1_ref[...] = jnp.zeros_like(q1_ref)

    s1_ref[...] += jnp.sum(h1.reshape(nb // 8, 8, h1.shape[1]), axis=0)
    q1_ref[...] += jnp.sum((h1 * h1).reshape(nb // 8, 8, h1.shape[1]), axis=0)


def _bn_gelu_stats_kernel(h_ref, s_ref, q_ref, g_ref, bt_ref, w_ref,
                          s2_ref, q2_ref, *, count, nb):
    i = pl.program_id(0)
    mean = jnp.sum(s_ref[...], axis=0, keepdims=True) / count
    var = jnp.sum(q_ref[...], axis=0, keepdims=True) / count - mean * mean
    scale = g_ref[...] * lax.rsqrt(var + 1e-5)
    x = h_ref[...].astype(jnp.float32)
    xb = (x - mean) * scale + bt_ref[...]
    g = _gelu(xb).astype(jnp.bfloat16)
    h2 = lax.dot(g, w_ref[...], preferred_element_type=jnp.float32)

    @pl.when(i == 0)
    def _init():
        s2_ref[...] = jnp.zeros_like(s2_ref)
        q2_ref[...] = jnp.zeros_like(q2_ref)

    s2_ref[...] += jnp.sum(h2.reshape(nb // 8, 8, h2.shape[1]), axis=0)
    q2_ref[...] += jnp.sum((h2 * h2).reshape(nb // 8, 8, h2.shape[1]), axis=0)


def _final_kernel(h_ref, s_ref, q_ref, g_ref, bt_ref, w_ref,
                  s2_ref, q2_ref, g2_ref, bt2_ref, out_ref, *, count):
    mean = jnp.sum(s_ref[...], axis=0, keepdims=True) / count
    var = jnp.sum(q_ref[...], axis=0, keepdims=True) / count - mean * mean
    scale = g_ref[...] * lax.rsqrt(var + 1e-5)
    x = h_ref[...].astype(jnp.float32)
    xb = (x - mean) * scale + bt_ref[...]
    g = _gelu(xb).astype(jnp.bfloat16)
    h2 = lax.dot(g, w_ref[...], preferred_element_type=jnp.float32)

    mean2 = jnp.sum(s2_ref[...], axis=0, keepdims=True) / count
    var2 = jnp.sum(q2_ref[...], axis=0, keepdims=True) / count - mean2 * mean2
    scale2 = g2_ref[...] * lax.rsqrt(var2 + 1e-5)
    out_ref[...] = _gelu((h2 - mean2) * scale2 + bt2_ref[...])


def kernel(super_xyz, super_point_features, xyz, point_features,
           W1, gamma1, beta1, W2, gamma2, beta2):
    B, S, F = super_point_features.shape
    N = xyz.shape[1]
    C = point_features.shape[2]
    H1 = W1.shape[1]
    H2 = W2.shape[1]
    NB = 512
    M = B * N
    NB2 = 2048

    sxyzt = jnp.transpose(super_xyz, (0, 2, 1))
    w1_top = W1[:C].astype(jnp.bfloat16)
    w1_bot = W1[C:]
    w2_b = W2.astype(jnp.bfloat16)

    h1, s1, q1 = pl.pallas_call(
        functools.partial(_topk_interp_kernel, nb=NB),
        grid=(B, N // NB),
        in_specs=[
            pl.BlockSpec((1, NB, 3), lambda b, n: (b, n, 0)),
            pl.BlockSpec((1, NB, C), lambda b, n: (b, n, 0)),
            pl.BlockSpec((1, 3, S), lambda b, n: (b, 0, 0)),
            pl.BlockSpec((1, S, F), lambda b, n: (b, 0, 0)),
            pl.BlockSpec((F, H1), lambda b, n: (0, 0)),
            pl.BlockSpec((C, H1), lambda b, n: (0, 0)),
        ],
        out_specs=[
            pl.BlockSpec((1, NB, H1), lambda b, n: (b, n, 0)),
            pl.BlockSpec((8, H1), lambda b, n: (0, 0)),
            pl.BlockSpec((8, H1), lambda b, n: (0, 0)),
        ],
        out_shape=[
            jax.ShapeDtypeStruct((B, N, H1), jnp.bfloat16),
            jax.ShapeDtypeStruct((8, H1), jnp.float32),
            jax.ShapeDtypeStruct((8, H1), jnp.float32),
        ],
        scratch_shapes=[pltpu.VMEM((S, H1), jnp.bfloat16)],
    )(xyz, point_features, sxyzt, super_point_features, w1_bot, w1_top)

    h1f = h1.reshape(M, H1)
    s2, q2 = pl.pallas_call(
        functools.partial(_bn_gelu_stats_kernel, count=float(M), nb=NB2),
        grid=(M // NB2,),
        in_specs=[
            pl.BlockSpec((NB2, H1), lambda i: (i, 0)),
            pl.BlockSpec((8, H1), lambda i: (0, 0)),
            pl.BlockSpec((8, H1), lambda i: (0, 0)),
            pl.BlockSpec((1, H1), lambda i: (0, 0)),
            pl.BlockSpec((1, H1), lambda i: (0, 0)),
            pl.BlockSpec((H1, H2), lambda i: (0, 0)),
        ],
        out_specs=[
            pl.BlockSpec((8, H2), lambda i: (0, 0)),
            pl.BlockSpec((8, H2), lambda i: (0, 0)),
        ],
        out_shape=[
            jax.ShapeDtypeStruct((8, H2), jnp.float32),
            jax.ShapeDtypeStruct((8, H2), jnp.float32),
        ],
    )(h1f, s1, q1, gamma1.reshape(1, H1), beta1.reshape(1, H1), w2_b)

    out = pl.pallas_call(
        functools.partial(_final_kernel, count=float(M)),
        grid=(M // NB2,),
        in_specs=[
            pl.BlockSpec((NB2, H1), lambda i: (i, 0)),
            pl.BlockSpec((8, H1), lambda i: (0, 0)),
            pl.BlockSpec((8, H1), lambda i: (0, 0)),
            pl.BlockSpec((1, H1), lambda i: (0, 0)),
            pl.BlockSpec((1, H1), lambda i: (0, 0)),
            pl.BlockSpec((H1, H2), lambda i: (0, 0)),
            pl.BlockSpec((8, H2), lambda i: (0, 0)),
            pl.BlockSpec((8, H2), lambda i: (0, 0)),
            pl.BlockSpec((1, H2), lambda i: (0, 0)),
            pl.BlockSpec((1, H2), lambda i: (0, 0)),
        ],
        out_specs=pl.BlockSpec((NB2, H2), lambda i: (i, 0)),
        out_shape=jax.ShapeDtypeStruct((M, H2), jnp.float32),
    )(h1f, s1, q1, gamma1.reshape(1, H1), beta1.reshape(1, H1), w2_b,
      s2, q2, gamma2.reshape(1, H2), beta2.reshape(1, H2))

    return out.reshape(B, N, H2)


# speedup vs baseline: 1.1107x; 1.0491x over previous
import functools
import jax
import jax.numpy as jnp
from jax import lax
from jax.experimental import pallas as pl
from jax.experimental.pallas import tpu as pltpu

_SQRT_HALF = 0.7071067811865476
_F32_EPS = float(jnp.finfo(jnp.float32).eps)


def _gelu(x):
    return 0.5 * x * (1.0 + lax.erf(x * _SQRT_HALF))


def _topk_interp_kernel(xyz_ref, pf_ref, sxyzt_ref, sfeat_ref, w1b_ref,
                        w1t_ref, h1_ref, s1_ref, q1_ref, p_scr, *, nb):
    b = pl.program_id(0)
    n = pl.program_id(1)

    @pl.when(n == 0)
    def _proj():
        p_scr[...] = lax.dot(
            sfeat_ref[0], w1b_ref[...],
            preferred_element_type=jnp.float32).astype(jnp.bfloat16)

    x = xyz_ref[0]
    st = sxyzt_ref[0]
    t = lax.dot(x.astype(jnp.bfloat16), st.astype(jnp.bfloat16),
                preferred_element_type=jnp.float32)
    xn = jnp.sum(x * x, axis=1, keepdims=True)
    dd = jnp.sum(st * st, axis=0, keepdims=True) - 2.0 * t

    wmat = jnp.zeros_like(dd)
    wsum = jnp.zeros((nb, 1), jnp.float32)
    for _ in range(3):
        m = jnp.min(dd, axis=1, keepdims=True)
        e = dd == m
        w = 1.0 / (jnp.maximum(m + xn, 0.0) + _F32_EPS)
        wmat = jnp.where(e, jnp.broadcast_to(w, dd.shape), wmat)
        wsum = wsum + w
        dd = jnp.where(e, jnp.float32(jnp.inf), dd)
    wmat = (wmat / wsum).astype(jnp.bfloat16)

    h1 = lax.dot(wmat, p_scr[...], preferred_element_type=jnp.float32)
    h1 = h1 + lax.dot(pf_ref[0].astype(jnp.bfloat16), w1t_ref[...],
                      preferred_element_type=jnp.float32)
    h1_ref[0] = h1.astype(jnp.bfloat16)

    @pl.when((b == 0) & (n == 0))
    def _init():
        s1_ref[...] = jnp.zeros_like(s1_ref)
        q1_ref[...] = jnp.zeros_like(q1_ref)

    s1_ref[...] += jnp.sum(h1.reshape(nb // 8, 8, h1.shape[1]), axis=0)
    q1_ref[...] += jnp.sum((h1 * h1).reshape(nb // 8, 8, h1.shape[1]), axis=0)


def _bn_gelu_stats_kernel(h_ref, s_ref, q_ref, g_ref, bt_ref, w_ref,
                          s2_ref, q2_ref, *, count, nb):
    i = pl.program_id(0)
    mean = jnp.sum(s_ref[...], axis=0, keepdims=True) / count
    var = jnp.sum(q_ref[...], axis=0, keepdims=True) / count - mean * mean
    scale = g_ref[...] * lax.rsqrt(var + 1e-5)
    x = h_ref[...].astype(jnp.float32)
    xb = (x - mean) * scale + bt_ref[...]
    g = _gelu(xb).astype(jnp.bfloat16)
    h2 = lax.dot(g, w_ref[...], preferred_element_type=jnp.float32)

    @pl.when(i == 0)
    def _init():
        s2_ref[...] = jnp.zeros_like(s2_ref)
        q2_ref[...] = jnp.zeros_like(q2_ref)

    s2_ref[...] += jnp.sum(h2.reshape(nb // 8, 8, h2.shape[1]), axis=0)
    q2_ref[...] += jnp.sum((h2 * h2).reshape(nb // 8, 8, h2.shape[1]), axis=0)


def _final_kernel(h_ref, s_ref, q_ref, g_ref, bt_ref, w_ref,
                  s2_ref, q2_ref, g2_ref, bt2_ref, out_ref, *, count):
    mean = jnp.sum(s_ref[...], axis=0, keepdims=True) / count
    var = jnp.sum(q_ref[...], axis=0, keepdims=True) / count - mean * mean
    scale = g_ref[...] * lax.rsqrt(var + 1e-5)
    x = h_ref[...].astype(jnp.float32)
    xb = (x - mean) * scale + bt_ref[...]
    g = _gelu(xb).astype(jnp.bfloat16)
    h2 = lax.dot(g, w_ref[...], preferred_element_type=jnp.float32)

    mean2 = jnp.sum(s2_ref[...], axis=0, keepdims=True) / count
    var2 = jnp.sum(q2_ref[...], axis=0, keepdims=True) / count - mean2 * mean2
    scale2 = g2_ref[...] * lax.rsqrt(var2 + 1e-5)
    out_ref[...] = _gelu((h2 - mean2) * scale2 + bt2_ref[...])


def kernel(super_xyz, super_point_features, xyz, point_features,
           W1, gamma1, beta1, W2, gamma2, beta2):
    B, S, F = super_point_features.shape
    N = xyz.shape[1]
    C = point_features.shape[2]
    H1 = W1.shape[1]
    H2 = W2.shape[1]
    NB = 1024
    M = B * N
    NB2 = 2048

    sxyzt = jnp.transpose(super_xyz, (0, 2, 1))
    w1_top = W1[:C].astype(jnp.bfloat16)
    w1_bot = W1[C:]
    w2_b = W2.astype(jnp.bfloat16)

    h1, s1, q1 = pl.pallas_call(
        functools.partial(_topk_interp_kernel, nb=NB),
        grid=(B, N // NB),
        in_specs=[
            pl.BlockSpec((1, NB, 3), lambda b, n: (b, n, 0)),
            pl.BlockSpec((1, NB, C), lambda b, n: (b, n, 0)),
            pl.BlockSpec((1, 3, S), lambda b, n: (b, 0, 0)),
            pl.BlockSpec((1, S, F), lambda b, n: (b, 0, 0)),
            pl.BlockSpec((F, H1), lambda b, n: (0, 0)),
            pl.BlockSpec((C, H1), lambda b, n: (0, 0)),
        ],
        out_specs=[
            pl.BlockSpec((1, NB, H1), lambda b, n: (b, n, 0)),
            pl.BlockSpec((8, H1), lambda b, n: (0, 0)),
            pl.BlockSpec((8, H1), lambda b, n: (0, 0)),
        ],
        out_shape=[
            jax.ShapeDtypeStruct((B, N, H1), jnp.bfloat16),
            jax.ShapeDtypeStruct((8, H1), jnp.float32),
            jax.ShapeDtypeStruct((8, H1), jnp.float32),
        ],
        scratch_shapes=[pltpu.VMEM((S, H1), jnp.bfloat16)],
    )(xyz, point_features, sxyzt, super_point_features, w1_bot, w1_top)

    h1f = h1.reshape(M, H1)
    s2, q2 = pl.pallas_call(
        functools.partial(_bn_gelu_stats_kernel, count=float(M), nb=NB2),
        grid=(M // NB2,),
        in_specs=[
            pl.BlockSpec((NB2, H1), lambda i: (i, 0)),
            pl.BlockSpec((8, H1), lambda i: (0, 0)),
            pl.BlockSpec((8, H1), lambda i: (0, 0)),
            pl.BlockSpec((1, H1), lambda i: (0, 0)),
            pl.BlockSpec((1, H1), lambda i: (0, 0)),
            pl.BlockSpec((H1, H2), lambda i: (0, 0)),
        ],
        out_specs=[
            pl.BlockSpec((8, H2), lambda i: (0, 0)),
            pl.BlockSpec((8, H2), lambda i: (0, 0)),
        ],
        out_shape=[
            jax.ShapeDtypeStruct((8, H2), jnp.float32),
            jax.ShapeDtypeStruct((8, H2), jnp.float32),
        ],
    )(h1f, s1, q1, gamma1.reshape(1, H1), beta1.reshape(1, H1), w2_b)

    out = pl.pallas_call(
        functools.partial(_final_kernel, count=float(M)),
        grid=(M // NB2,),
        in_specs=[
            pl.BlockSpec((NB2, H1), lambda i: (i, 0)),
            pl.BlockSpec((8, H1), lambda i: (0, 0)),
            pl.BlockSpec((8, H1), lambda i: (0, 0)),
            pl.BlockSpec((1, H1), lambda i: (0, 0)),
            pl.BlockSpec((1, H1), lambda i: (0, 0)),
            pl.BlockSpec((H1, H2), lambda i: (0, 0)),
            pl.BlockSpec((8, H2), lambda i: (0, 0)),
            pl.BlockSpec((8, H2), lambda i: (0, 0)),
            pl.BlockSpec((1, H2), lambda i: (0, 0)),
            pl.BlockSpec((1, H2), lambda i: (0, 0)),
        ],
        out_specs=pl.BlockSpec((NB2, H2), lambda i: (i, 0)),
        out_shape=jax.ShapeDtypeStruct((M, H2), jnp.float32),
    )(h1f, s1, q1, gamma1.reshape(1, H1), beta1.reshape(1, H1), w2_b,
      s2, q2, gamma2.reshape(1, H2), beta2.reshape(1, H2))

    return out.reshape(B, N, H2)
